# baseline (speedup 1.0000x reference)
import functools

import jax
import jax.numpy as jnp
from jax import lax
from jax.experimental import pallas as pl
from jax.experimental.pallas import tpu as pltpu
from jax.experimental.pallas import tpu_sc as plsc

N = 10000
E = 320000
NFEAT = 128
NHID = 128
NCLASS = 40
NCLS_PAD = 64

NC = 2
NS = 16
NW = NC * NS
CHUNK = 128
NCHUNKS = 80
EPT = NCHUNKS * CHUNK
EPAD = NW * EPT
STRIPE = 624
TAIL_OFF = NS * STRIPE
TAIL = N - TAIL_OFF


def _make_spmm(D: int, NB: int):
  NV = D // 16
  mesh = plsc.VectorSubcoreMesh(core_axis_name="c", subcore_axis_name="s")

  @functools.partial(
      pl.kernel,
      out_type=jax.ShapeDtypeStruct((NC, N, D), jnp.float32),
      mesh=mesh,
      compiler_params=pltpu.CompilerParams(use_tc_tiling_on_sc=(D % 128 == 0)),
      scratch_types=[
          pltpu.VMEM((NCHUNKS, CHUNK), jnp.int32),
          pltpu.VMEM_SHARED((N, D), jnp.float32),
      ] + [pltpu.VMEM((CHUNK, D), jnp.float32) for _ in range(NB)]
        + [pltpu.VMEM((2, CHUNK), jnp.int32) for _ in range(NB)]
        + [pltpu.VMEM((CHUNK,), jnp.int32) for _ in range(NB)]
        + [pltpu.SemaphoreType.DMA for _ in range(3 * NB)],
  )
  def spmm(sup_hbm, rw_hbm, col_hbm, zero_hbm, out_hbm, colv, acc, *bufs):
    gbufs = bufs[:NB]
    rwbufs = bufs[NB:2 * NB]
    ribufs = bufs[2 * NB:3 * NB]
    gsems = bufs[3 * NB:4 * NB]
    ssems = bufs[4 * NB:5 * NB]
    rwsems = bufs[5 * NB:]
    c = lax.axis_index("c")
    s = lax.axis_index("s")
    wid = c * NS + s

    pltpu.sync_copy(col_hbm.at[wid], colv)

    pltpu.sync_copy(zero_hbm.at[pl.ds(s * STRIPE, STRIPE)],
                    acc.at[pl.ds(s * STRIPE, STRIPE)])

    @pl.when(s == NS - 1)
    def _zero_tail():
      pltpu.sync_copy(zero_hbm.at[pl.ds(TAIL_OFF, TAIL)],
                      acc.at[pl.ds(TAIL_OFF, TAIL)])

    plsc.subcore_barrier()

    for b in range(NB):
      pltpu.async_copy(rw_hbm.at[wid, b], rwbufs[b], rwsems[b])
      pltpu.async_copy(sup_hbm.at[colv.at[b]], gbufs[b], gsems[b])

    @pl.loop(0, NCHUNKS // NB)
    def _iter(i):
      for b in range(NB):
        ci = i * NB + b
        gbuf, rwbuf, ribuf = gbufs[b], rwbufs[b], ribufs[b]
        gsem, ssem, rwsem = gsems[b], ssems[b], rwsems[b]
        bp = (b - 1) % NB

        pltpu.make_async_copy(sup_hbm.at[colv.at[ci]], gbuf, gsem).wait()
        pltpu.make_async_copy(rw_hbm.at[wid, ci], rwbuf, rwsem).wait()

        @pl.loop(0, CHUNK, step=16)
        def _grp(g):
          w16 = lax.bitcast_convert_type(rwbuf[1, pl.ds(g, 16)], jnp.float32)
          for l in range(16):
            we = w16[l]
            for j in range(NV):
              sl = pl.ds(j * 16, 16)
              gbuf[g + l, sl] = gbuf[g + l, sl] * we

        for k in range(CHUNK // 16):
          ks = pl.ds(k * 16, 16)
          ribuf[ks] = rwbuf[0, ks]
        pltpu.async_copy(gbuf, acc.at[ribuf], ssem, add=True)

        @pl.when(ci + NB < NCHUNKS)
        def _():
          pltpu.async_copy(rw_hbm.at[wid, ci + NB], rwbuf, rwsem)

        @pl.when((ci >= 1) & (ci - 1 + NB < NCHUNKS))
        def _():
          pltpu.make_async_copy(gbufs[bp], acc.at[ribufs[bp]], ssems[bp]).wait()
          pltpu.async_copy(sup_hbm.at[colv.at[ci - 1 + NB]], gbufs[bp],
                           gsems[bp])

    for b in range(NB):
      pltpu.make_async_copy(gbufs[b], acc.at[ribufs[b]], ssems[b]).wait()

    plsc.subcore_barrier()
    pltpu.sync_copy(acc.at[pl.ds(s * STRIPE, STRIPE)],
                    out_hbm.at[c, pl.ds(s * STRIPE, STRIPE)])

    @pl.when(s == NS - 1)
    def _write_tail():
      pltpu.sync_copy(acc.at[pl.ds(TAIL_OFF, TAIL)],
                      out_hbm.at[c, pl.ds(TAIL_OFF, TAIL)])

  return spmm


_spmm_hid = _make_spmm(NHID, 2)
_spmm_cls = _make_spmm(NCLS_PAD, 4)


_RB = 1000


def _mm_body(x_ref, w_ref, o_ref):
  o_ref[...] = jnp.dot(x_ref[...], w_ref[...],
                       preferred_element_type=jnp.float32,
                       precision=lax.Precision.HIGHEST)


def _tc_matmul(x, w):
  n, k = x.shape
  m = w.shape[1]
  return pl.pallas_call(
      _mm_body,
      grid=(n // _RB,),
      in_specs=[pl.BlockSpec((_RB, k), lambda i: (i, 0)),
                pl.BlockSpec((k, m), lambda i: (0, 0))],
      out_specs=pl.BlockSpec((_RB, m), lambda i: (i, 0)),
      out_shape=jax.ShapeDtypeStruct((n, m), jnp.float32),
  )(x, w)


def _fuse_body(p_ref, b_ref, w_ref, o_ref):
  h = p_ref[0] + p_ref[1] + b_ref[...]
  h = jnp.maximum(h, 0.0)
  o_ref[...] = jnp.dot(h, w_ref[...],
                       preferred_element_type=jnp.float32,
                       precision=lax.Precision.HIGHEST)


def _tc_fuse(p, b_row, w):
  k = p.shape[2]
  m = w.shape[1]
  return pl.pallas_call(
      _fuse_body,
      grid=(N // _RB,),
      in_specs=[pl.BlockSpec((NC, _RB, k), lambda i: (0, i, 0)),
                pl.BlockSpec((1, k), lambda i: (0, 0)),
                pl.BlockSpec((k, m), lambda i: (0, 0))],
      out_specs=pl.BlockSpec((_RB, m), lambda i: (i, 0)),
      out_shape=jax.ShapeDtypeStruct((N, m), jnp.float32),
  )(p, b_row, w)


def _final_body(q_ref, b_ref, o_ref):
  z = q_ref[0] + q_ref[1] + b_ref[...]
  z = z[:, :NCLASS]
  m = jnp.max(z, axis=1, keepdims=True)
  lse = jnp.log(jnp.sum(jnp.exp(z - m), axis=1, keepdims=True)) + m
  o_ref[...] = z - lse


def _tc_final(q, b_row):
  return pl.pallas_call(
      _final_body,
      grid=(N // _RB,),
      in_specs=[pl.BlockSpec((NC, _RB, NCLS_PAD), lambda i: (0, i, 0)),
                pl.BlockSpec((1, NCLS_PAD), lambda i: (0, 0))],
      out_specs=pl.BlockSpec((_RB, NCLASS), lambda i: (i, 0)),
      out_shape=jax.ShapeDtypeStruct((N, NCLASS), jnp.float32),
  )(q, b_row)


def kernel(x, edge_index, edge_weight, W1, b1, W2, b2):
  row = edge_index[0].astype(jnp.int32)
  col = edge_index[1].astype(jnp.int32)
  w = edge_weight.astype(jnp.float32)
  perm = jnp.argsort(col)
  row, col, w = row[perm], col[perm], w[perm]
  pad = EPAD - E
  row3 = jnp.concatenate([row, jnp.zeros((pad,), jnp.int32)]).reshape(
      NW, NCHUNKS, CHUNK)
  col3 = jnp.concatenate([col, jnp.zeros((pad,), jnp.int32)]).reshape(
      NW, NCHUNKS, CHUNK)
  wbits = lax.bitcast_convert_type(
      jnp.concatenate([w, jnp.zeros((pad,), jnp.float32)]), jnp.int32
  ).reshape(NW, NCHUNKS, CHUNK)
  rw4 = jnp.stack([row3, wbits], axis=2)

  zeros_hid = jnp.zeros((N, NHID), jnp.float32)
  zeros_cls = jnp.zeros((N, NCLS_PAD), jnp.float32)
  W2p = jnp.zeros((NHID, NCLS_PAD), jnp.float32).at[:, :NCLASS].set(W2)
  b1_row = b1.reshape(1, NHID)
  b2_row = jnp.zeros((1, NCLS_PAD), jnp.float32).at[0, :NCLASS].set(b2)

  support1 = _tc_matmul(x, W1)
  p = _spmm_hid(support1, rw4, col3, zeros_hid)
  support2 = _tc_fuse(p, b1_row, W2p)
  q = _spmm_cls(support2, rw4, col3, zeros_cls)
  return _tc_final(q, b2_row)


# speedup vs baseline: 2.6375x; 2.6375x over previous
import functools

import jax
import jax.numpy as jnp
from jax import lax
from jax.experimental import pallas as pl
from jax.experimental.pallas import tpu as pltpu
from jax.experimental.pallas import tpu_sc as plsc

N = 10000
E = 320000
NFEAT = 128
NHID = 128
NCLASS = 40
NCLS_PAD = 64

NC = 2
NS = 16
NW = NC * NS
CHUNK = 128
NB = 2
NCOL = 4
NCHUNKS = 80
EPT = NCHUNKS * CHUNK
EPAD = NW * EPT
STRIPE = 624
TAIL_OFF = NS * STRIPE
TAIL = N - TAIL_OFF


def _make_spmm(D: int):
  DH = D // 2
  mesh = plsc.VectorSubcoreMesh(core_axis_name="c", subcore_axis_name="s")

  @functools.partial(
      pl.kernel,
      out_type=jax.ShapeDtypeStruct((NC, N, D), jnp.float32),
      mesh=mesh,
      compiler_params=pltpu.CompilerParams(use_tc_tiling_on_sc=False),
      scratch_types=[
          pltpu.VMEM_SHARED((N, D), jnp.float32),
      ] + [pltpu.VMEM((CHUNK, DH), jnp.int32) for _ in range(NB)]
        + [pltpu.VMEM((CHUNK, D), jnp.float32) for _ in range(NB)]
        + [pltpu.VMEM((2, CHUNK), jnp.int32) for _ in range(NB)]
        + [pltpu.VMEM((CHUNK,), jnp.int32) for _ in range(NB)]
        + [pltpu.VMEM((CHUNK,), jnp.int32) for _ in range(NCOL)]
        + [pltpu.SemaphoreType.DMA for _ in range(3 * NB + NCOL)],
  )
  def spmm(sup_hbm, rw_hbm, col_hbm, zero_hbm, out_hbm, acc, *bufs):
    gbufs = bufs[:NB]
    sbufs = bufs[NB:2 * NB]
    rwbufs = bufs[2 * NB:3 * NB]
    ribufs = bufs[3 * NB:4 * NB]
    colbufs = bufs[4 * NB:4 * NB + NCOL]
    sems = bufs[4 * NB + NCOL:]
    gsems = sems[:NB]
    ssems = sems[NB:2 * NB]
    rwsems = sems[2 * NB:3 * NB]
    csems = sems[3 * NB:]
    c = lax.axis_index("c")
    s = lax.axis_index("s")
    wid = c * NS + s

    pltpu.sync_copy(zero_hbm.at[pl.ds(s * STRIPE, STRIPE)],
                    acc.at[pl.ds(s * STRIPE, STRIPE)])

    @pl.when(s == NS - 1)
    def _zero_tail():
      pltpu.sync_copy(zero_hbm.at[pl.ds(TAIL_OFF, TAIL)],
                      acc.at[pl.ds(TAIL_OFF, TAIL)])

    plsc.subcore_barrier()

    for u in range(NCOL):
      pltpu.async_copy(col_hbm.at[wid, u], colbufs[u], csems[u])
    for b in range(NB):
      pltpu.make_async_copy(col_hbm.at[wid, b], colbufs[b], csems[b]).wait()
      pltpu.async_copy(sup_hbm.at[colbufs[b]], gbufs[b], gsems[b])
      pltpu.async_copy(rw_hbm.at[wid, b], rwbufs[b], rwsems[b])

    @pl.loop(0, NCHUNKS // NCOL)
    def _iter(i):
      for u in range(NCOL):
        ci = i * NCOL + u
        g = u % NB
        un = (u + NB) % NCOL
        gbuf, sbuf, rwbuf, ribuf = gbufs[g], sbufs[g], rwbufs[g], ribufs[g]
        gsem, ssem, rwsem = gsems[g], ssems[g], rwsems[g]

        pltpu.make_async_copy(sup_hbm.at[colbufs[u]], gbuf, gsem).wait()
        pltpu.make_async_copy(rw_hbm.at[wid, ci], rwbuf, rwsem).wait()

        @pl.when(ci >= NB)
        def _():
          pltpu.make_async_copy(sbuf, acc.at[ribuf], ssem).wait()

        @pl.loop(0, CHUNK, step=16)
        def _grp(gg):
          w16 = lax.bitcast_convert_type(rwbuf[1, pl.ds(gg, 16)], jnp.float32)
          for l in range(16):
            we = w16[l]
            for j in range(DH // 16):
              pk = gbuf[gg + l, pl.ds(j * 16, 16)]
              lo = lax.bitcast_convert_type(pk << 16, jnp.float32)
              hi = lax.bitcast_convert_type(pk & jnp.int32(-65536),
                                            jnp.float32)
              sbuf[gg + l, pl.ds(j * 16, 16)] = lo * we
              sbuf[gg + l, pl.ds(DH + j * 16, 16)] = hi * we

        for k in range(CHUNK // 16):
          ks = pl.ds(k * 16, 16)
          ribuf[ks] = rwbuf[0, ks]

        @pl.when(ci + NCOL < NCHUNKS)
        def _():
          pltpu.async_copy(col_hbm.at[wid, ci + NCOL], colbufs[u], csems[u])

        @pl.when(ci + NB < NCHUNKS)
        def _():
          pltpu.make_async_copy(col_hbm.at[wid, ci + NB], colbufs[un],
                                csems[un]).wait()
          pltpu.async_copy(sup_hbm.at[colbufs[un]], gbuf, gsem)
          pltpu.async_copy(rw_hbm.at[wid, ci + NB], rwbuf, rwsem)

        pltpu.async_copy(sbuf, acc.at[ribuf], ssem, add=True)

    for b in range(NB):
      pltpu.make_async_copy(sbufs[b], acc.at[ribufs[b]], ssems[b]).wait()

    plsc.subcore_barrier()
    pltpu.sync_copy(acc.at[pl.ds(s * STRIPE, STRIPE)],
                    out_hbm.at[c, pl.ds(s * STRIPE, STRIPE)])

    @pl.when(s == NS - 1)
    def _write_tail():
      pltpu.sync_copy(acc.at[pl.ds(TAIL_OFF, TAIL)],
                      out_hbm.at[c, pl.ds(TAIL_OFF, TAIL)])

  return spmm


_spmm_hid = _make_spmm(NHID)
_spmm_cls = _make_spmm(NCLS_PAD)


_RB = 1000


def _pack_rows(res):
  dh = res.shape[1] // 2
  lo = lax.bitcast_convert_type(res[:, :dh].astype(jnp.bfloat16),
                                jnp.uint16).astype(jnp.int32)
  hi = lax.bitcast_convert_type(res[:, dh:].astype(jnp.bfloat16),
                                jnp.uint16).astype(jnp.int32)
  return lo | (hi << 16)


def _mm_body(x_ref, w_ref, o_ref):
  o_ref[...] = _pack_rows(jnp.dot(x_ref[...], w_ref[...],
                                  preferred_element_type=jnp.float32,
                                  precision=lax.Precision.HIGHEST))


def _tc_matmul(x, w):
  n, k = x.shape
  m = w.shape[1]
  return pl.pallas_call(
      _mm_body,
      grid=(n // _RB,),
      in_specs=[pl.BlockSpec((_RB, k), lambda i: (i, 0)),
                pl.BlockSpec((k, m), lambda i: (0, 0))],
      out_specs=pl.BlockSpec((_RB, m // 2), lambda i: (i, 0)),
      out_shape=jax.ShapeDtypeStruct((n, m // 2), jnp.int32),
  )(x, w)


def _fuse_body(p_ref, b_ref, w_ref, o_ref):
  h = p_ref[0] + p_ref[1] + b_ref[...]
  h = jnp.maximum(h, 0.0)
  o_ref[...] = _pack_rows(jnp.dot(h, w_ref[...],
                                  preferred_element_type=jnp.float32,
                                  precision=lax.Precision.HIGHEST))


def _tc_fuse(p, b_row, w):
  k = p.shape[2]
  m = w.shape[1]
  return pl.pallas_call(
      _fuse_body,
      grid=(N // _RB,),
      in_specs=[pl.BlockSpec((NC, _RB, k), lambda i: (0, i, 0)),
                pl.BlockSpec((1, k), lambda i: (0, 0)),
                pl.BlockSpec((k, m), lambda i: (0, 0))],
      out_specs=pl.BlockSpec((_RB, m // 2), lambda i: (i, 0)),
      out_shape=jax.ShapeDtypeStruct((N, m // 2), jnp.int32),
  )(p, b_row, w)


def _final_body(q_ref, b_ref, o_ref):
  z = q_ref[0] + q_ref[1] + b_ref[...]
  z = z[:, :NCLASS]
  m = jnp.max(z, axis=1, keepdims=True)
  lse = jnp.log(jnp.sum(jnp.exp(z - m), axis=1, keepdims=True)) + m
  o_ref[...] = z - lse


def _tc_final(q, b_row):
  return pl.pallas_call(
      _final_body,
      grid=(N // _RB,),
      in_specs=[pl.BlockSpec((NC, _RB, NCLS_PAD), lambda i: (0, i, 0)),
                pl.BlockSpec((1, NCLS_PAD), lambda i: (0, 0))],
      out_specs=pl.BlockSpec((_RB, NCLASS), lambda i: (i, 0)),
      out_shape=jax.ShapeDtypeStruct((N, NCLASS), jnp.float32),
  )(q, b_row)


def kernel(x, edge_index, edge_weight, W1, b1, W2, b2):
  row = edge_index[0].astype(jnp.int32)
  col = edge_index[1].astype(jnp.int32)
  w = edge_weight.astype(jnp.float32)
  pad = EPAD - E
  row3 = jnp.concatenate([row, jnp.zeros((pad,), jnp.int32)]).reshape(
      NW, NCHUNKS, CHUNK)
  col3 = jnp.concatenate([col, jnp.zeros((pad,), jnp.int32)]).reshape(
      NW, NCHUNKS, CHUNK)
  wbits = lax.bitcast_convert_type(
      jnp.concatenate([w, jnp.zeros((pad,), jnp.float32)]), jnp.int32
  ).reshape(NW, NCHUNKS, CHUNK)
  rw4 = jnp.stack([row3, wbits], axis=2)

  zeros_hid = jnp.zeros((N, NHID), jnp.float32)
  zeros_cls = jnp.zeros((N, NCLS_PAD), jnp.float32)
  W2p = jnp.zeros((NHID, NCLS_PAD), jnp.float32).at[:, :NCLASS].set(W2)
  b1_row = b1.reshape(1, NHID)
  b2_row = jnp.zeros((1, NCLS_PAD), jnp.float32).at[0, :NCLASS].set(b2)

  support1 = _tc_matmul(x, W1)
  p = _spmm_hid(support1, rw4, col3, zeros_hid)
  support2 = _tc_fuse(p, b1_row, W2p)
  q = _spmm_cls(support2, rw4, col3, zeros_cls)
  return _tc_final(q, b2_row)


# speedup vs baseline: 2.6759x; 1.0145x over previous
import functools

import jax
import jax.numpy as jnp
from jax import lax
from jax.experimental import pallas as pl
from jax.experimental.pallas import tpu as pltpu
from jax.experimental.pallas import tpu_sc as plsc

N = 10000
E = 320000
NFEAT = 128
NHID = 128
NCLASS = 40
NCLS_PAD = 64

NC = 2
NS = 16
NW = NC * NS
CHUNK = 128
NB = 2
NCOL = 4
NCHUNKS = 80
EPT = NCHUNKS * CHUNK
EPAD = NW * EPT
STRIPE = 624
TAIL_OFF = NS * STRIPE
TAIL = N - TAIL_OFF


def _make_spmm(D: int):
  DH = D // 2
  mesh = plsc.VectorSubcoreMesh(core_axis_name="c", subcore_axis_name="s")

  @functools.partial(
      pl.kernel,
      out_type=jax.ShapeDtypeStruct((NC, N, D), jnp.float32),
      mesh=mesh,
      compiler_params=pltpu.CompilerParams(use_tc_tiling_on_sc=False),
      scratch_types=[
          pltpu.VMEM_SHARED((N, D), jnp.float32),
      ] + [pltpu.VMEM((CHUNK, DH), jnp.int32) for _ in range(NB)]
        + [pltpu.VMEM((CHUNK, D), jnp.float32) for _ in range(NB)]
        + [pltpu.VMEM((2, CHUNK), jnp.int32) for _ in range(NB)]
        + [pltpu.VMEM((CHUNK,), jnp.int32) for _ in range(NB)]
        + [pltpu.VMEM((CHUNK,), jnp.int32) for _ in range(NCOL)]
        + [pltpu.SemaphoreType.DMA for _ in range(3 * NB + NCOL)],
  )
  def spmm(sup_hbm, rw_hbm, col_hbm, zero_hbm, out_hbm, acc, *bufs):
    gbufs = bufs[:NB]
    sbufs = bufs[NB:2 * NB]
    rwbufs = bufs[2 * NB:3 * NB]
    ribufs = bufs[3 * NB:4 * NB]
    colbufs = bufs[4 * NB:4 * NB + NCOL]
    sems = bufs[4 * NB + NCOL:]
    gsems = sems[:NB]
    ssems = sems[NB:2 * NB]
    rwsems = sems[2 * NB:3 * NB]
    csems = sems[3 * NB:]
    c = lax.axis_index("c")
    s = lax.axis_index("s")
    wid = c * NS + s

    pltpu.sync_copy(zero_hbm.at[pl.ds(s * STRIPE, STRIPE)],
                    acc.at[pl.ds(s * STRIPE, STRIPE)])

    @pl.when(s == NS - 1)
    def _zero_tail():
      pltpu.sync_copy(zero_hbm.at[pl.ds(TAIL_OFF, TAIL)],
                      acc.at[pl.ds(TAIL_OFF, TAIL)])

    plsc.subcore_barrier()

    for u in range(NCOL):
      pltpu.async_copy(col_hbm.at[wid, u], colbufs[u], csems[u])
    for b in range(NB):
      pltpu.make_async_copy(col_hbm.at[wid, b], colbufs[b], csems[b]).wait()
      pltpu.async_copy(sup_hbm.at[colbufs[b]], gbufs[b], gsems[b])
      pltpu.async_copy(rw_hbm.at[wid, b], rwbufs[b], rwsems[b])

    @pl.loop(0, NCHUNKS // NCOL)
    def _iter(i):
      for u in range(NCOL):
        ci = i * NCOL + u
        g = u % NB
        un = (u + NB) % NCOL
        gbuf, sbuf, rwbuf, ribuf = gbufs[g], sbufs[g], rwbufs[g], ribufs[g]
        gsem, ssem, rwsem = gsems[g], ssems[g], rwsems[g]

        pltpu.make_async_copy(sup_hbm.at[colbufs[u]], gbuf, gsem).wait()
        pltpu.make_async_copy(rw_hbm.at[wid, ci], rwbuf, rwsem).wait()

        @pl.when(ci >= NB)
        def _():
          pltpu.make_async_copy(sbuf, acc.at[ribuf], ssem).wait()

        @pl.loop(0, CHUNK, step=16)
        def _grp(gg):
          w16 = lax.bitcast_convert_type(rwbuf[1, pl.ds(gg, 16)], jnp.float32)
          for l in range(16):
            we = w16[l]
            for j in range(DH // 16):
              pk = gbuf[gg + l, pl.ds(j * 16, 16)]
              lo = lax.bitcast_convert_type(pk << 16, jnp.float32)
              hi = lax.bitcast_convert_type(pk & jnp.int32(-65536),
                                            jnp.float32)
              sbuf[gg + l, pl.ds(j * 16, 16)] = lo * we
              sbuf[gg + l, pl.ds(DH + j * 16, 16)] = hi * we

        for k in range(CHUNK // 16):
          ks = pl.ds(k * 16, 16)
          ribuf[ks] = rwbuf[0, ks]

        @pl.when(ci + NCOL < NCHUNKS)
        def _():
          pltpu.async_copy(col_hbm.at[wid, ci + NCOL], colbufs[u], csems[u])

        @pl.when(ci + NB < NCHUNKS)
        def _():
          pltpu.make_async_copy(col_hbm.at[wid, ci + NB], colbufs[un],
                                csems[un]).wait()
          pltpu.async_copy(sup_hbm.at[colbufs[un]], gbuf, gsem)
          pltpu.async_copy(rw_hbm.at[wid, ci + NB], rwbuf, rwsem)

        pltpu.async_copy(sbuf, acc.at[ribuf], ssem, add=True)

    for b in range(NB):
      pltpu.make_async_copy(sbufs[b], acc.at[ribufs[b]], ssems[b]).wait()

    plsc.subcore_barrier()
    pltpu.sync_copy(acc.at[pl.ds(s * STRIPE, STRIPE)],
                    out_hbm.at[c, pl.ds(s * STRIPE, STRIPE)])

    @pl.when(s == NS - 1)
    def _write_tail():
      pltpu.sync_copy(acc.at[pl.ds(TAIL_OFF, TAIL)],
                      out_hbm.at[c, pl.ds(TAIL_OFF, TAIL)])

  return spmm


_spmm_hid = _make_spmm(NHID)
_spmm_cls = _make_spmm(NCLS_PAD)


_RB = 1000


def _pack_rows(res):
  dh = res.shape[1] // 2
  lo = lax.bitcast_convert_type(res[:, :dh].astype(jnp.bfloat16),
                                jnp.uint16).astype(jnp.int32)
  hi = lax.bitcast_convert_type(res[:, dh:].astype(jnp.bfloat16),
                                jnp.uint16).astype(jnp.int32)
  return lo | (hi << 16)


def _mm_body(x_ref, w_ref, o_ref):
  o_ref[...] = _pack_rows(jnp.dot(x_ref[...], w_ref[...],
                                  preferred_element_type=jnp.float32,
                                  precision=lax.Precision.DEFAULT))


def _tc_matmul(x, w):
  n, k = x.shape
  m = w.shape[1]
  return pl.pallas_call(
      _mm_body,
      grid=(n // _RB,),
      in_specs=[pl.BlockSpec((_RB, k), lambda i: (i, 0)),
                pl.BlockSpec((k, m), lambda i: (0, 0))],
      out_specs=pl.BlockSpec((_RB, m // 2), lambda i: (i, 0)),
      out_shape=jax.ShapeDtypeStruct((n, m // 2), jnp.int32),
  )(x, w)


def _fuse_body(p_ref, b_ref, w_ref, o_ref):
  h = p_ref[0] + p_ref[1] + b_ref[...]
  h = jnp.maximum(h, 0.0)
  o_ref[...] = _pack_rows(jnp.dot(h, w_ref[...],
                                  preferred_element_type=jnp.float32,
                                  precision=lax.Precision.DEFAULT))


def _tc_fuse(p, b_row, w):
  k = p.shape[2]
  m = w.shape[1]
  return pl.pallas_call(
      _fuse_body,
      grid=(N // _RB,),
      in_specs=[pl.BlockSpec((NC, _RB, k), lambda i: (0, i, 0)),
                pl.BlockSpec((1, k), lambda i: (0, 0)),
                pl.BlockSpec((k, m), lambda i: (0, 0))],
      out_specs=pl.BlockSpec((_RB, m // 2), lambda i: (i, 0)),
      out_shape=jax.ShapeDtypeStruct((N, m // 2), jnp.int32),
  )(p, b_row, w)


def _final_body(q_ref, b_ref, o_ref):
  z = q_ref[0] + q_ref[1] + b_ref[...]
  z = z[:, :NCLASS]
  m = jnp.max(z, axis=1, keepdims=True)
  lse = jnp.log(jnp.sum(jnp.exp(z - m), axis=1, keepdims=True)) + m
  o_ref[...] = z - lse


def _tc_final(q, b_row):
  return pl.pallas_call(
      _final_body,
      grid=(N // _RB,),
      in_specs=[pl.BlockSpec((NC, _RB, NCLS_PAD), lambda i: (0, i, 0)),
                pl.BlockSpec((1, NCLS_PAD), lambda i: (0, 0))],
      out_specs=pl.BlockSpec((_RB, NCLASS), lambda i: (i, 0)),
      out_shape=jax.ShapeDtypeStruct((N, NCLASS), jnp.float32),
  )(q, b_row)


def kernel(x, edge_index, edge_weight, W1, b1, W2, b2):
  row = edge_index[0].astype(jnp.int32)
  col = edge_index[1].astype(jnp.int32)
  w = edge_weight.astype(jnp.float32)
  pad = EPAD - E
  row3 = jnp.concatenate([row, jnp.zeros((pad,), jnp.int32)]).reshape(
      NW, NCHUNKS, CHUNK)
  col3 = jnp.concatenate([col, jnp.zeros((pad,), jnp.int32)]).reshape(
      NW, NCHUNKS, CHUNK)
  wbits = lax.bitcast_convert_type(
      jnp.concatenate([w, jnp.zeros((pad,), jnp.float32)]), jnp.int32
  ).reshape(NW, NCHUNKS, CHUNK)
  rw4 = jnp.stack([row3, wbits], axis=2)

  zeros_hid = jnp.zeros((N, NHID), jnp.float32)
  zeros_cls = jnp.zeros((N, NCLS_PAD), jnp.float32)
  W2p = jnp.zeros((NHID, NCLS_PAD), jnp.float32).at[:, :NCLASS].set(W2)
  b1_row = b1.reshape(1, NHID)
  b2_row = jnp.zeros((1, NCLS_PAD), jnp.float32).at[0, :NCLASS].set(b2)

  support1 = _tc_matmul(x, W1)
  p = _spmm_hid(support1, rw4, col3, zeros_hid)
  support2 = _tc_fuse(p, b1_row, W2p)
  q = _spmm_cls(support2, rw4, col3, zeros_cls)
  return _tc_final(q, b2_row)
